# baseline (speedup 1.0000x reference)
import functools
import math

import jax
import jax.numpy as jnp
from jax import lax
from jax.experimental import pallas as pl
from jax.experimental.pallas import tpu as pltpu
from jax.experimental.pallas import tpu_sc as plsc

B, C, H, W = 2, 4, 512, 512
N = H * W
NC, NS, L = 2, 16, 16
NW = NC * NS
NBIN = 5
ACC = (1 + C) * NBIN * L
LOG10 = math.log(10.0)

H_SC = 128
GB = 128

_mesh = plsc.VectorSubcoreMesh(
    core_axis_name="c", subcore_axis_name="s", num_cores=NC, num_subcores=NS
)


@functools.partial(
    pl.kernel,
    out_type=jax.ShapeDtypeStruct((NW, ACC), jnp.float32),
    mesh=_mesh,
    scratch_types=[
        pltpu.VMEM((2, H_SC // NS, W), jnp.int32),
        pltpu.VMEM((2, C, H_SC // NS, W), jnp.float32),
        pltpu.VMEM((NBIN * L,), jnp.float32),
        pltpu.VMEM((NBIN * L,), jnp.float32),
        pltpu.VMEM((NBIN * L,), jnp.float32),
        pltpu.VMEM((NBIN * L,), jnp.float32),
        pltpu.VMEM((NBIN * L,), jnp.float32),
        pltpu.VMEM((ACC,), jnp.float32),
        pltpu.SemaphoreType.DMA,
        pltpu.SemaphoreType.DMA,
    ],
    compiler_params=pltpu.CompilerParams(needs_layout_passes=False),
)
def _binned_sums(pred_hbm, lab_hbm, out_hbm, labv, chv, cnt, a0, a1, a2, a3,
                 stage, sem0, sem1):
    cid = lax.axis_index("c")
    sid = lax.axis_index("s")
    batch = cid
    rows = H_SC // NS
    NCHK = 1
    CR = rows // NCHK
    r0 = sid * rows
    accs = [cnt, a0, a1, a2, a3]
    sems = [sem0, sem1]

    zeros = jnp.zeros((L,), jnp.float32)
    for a in accs:
        for j in range(NBIN):
            a[pl.ds(j * L, L)] = zeros

    iota = lax.iota(jnp.int32, L)
    ones = jnp.ones((L,), jnp.float32)
    vecs_per_row = W // L

    def issue(g):
        buf = g & 1
        rb = r0 + g * CR
        hl = pltpu.async_copy(lab_hbm.at[batch, 0, pl.ds(rb, CR), :],
                              labv.at[buf], sems[buf])
        hp = pltpu.async_copy(pred_hbm.at[batch, :, pl.ds(rb, CR), :],
                              chv.at[buf], sems[buf])
        return (hl, hp)

    handles = issue(0)
    for g in range(NCHK):
        nxt = issue(g + 1) if g + 1 < NCHK else None
        for h in handles:
            h.wait()
        handles = nxt
        buf = g & 1

        @plsc.parallel_loop(0, CR * W // L, unroll=8)
        def _(i):
            r = i // vecs_per_row
            col = (i % vecs_per_row) * L
            lab = labv[buf, r, pl.ds(col, L)]
            idx = lab * L + iota
            plsc.addupdate_scatter(cnt, [idx], ones)
            for ch in range(C):
                x = chv[buf, ch, r, pl.ds(col, L)]
                plsc.addupdate_scatter(accs[1 + ch], [idx], x)

    wid = cid * NS + sid
    for j, a in enumerate(accs):
        for k in range(NBIN):
            stage[pl.ds((j * NBIN + k) * L, L)] = a[pl.ds(k * L, L)]
    pltpu.sync_copy(stage, out_hbm.at[wid])


KSTEP = (H - H_SC) // GB


def _tc_partial(pred_ref, lab_ref, out_ref):
    b = pl.program_id(0)
    k = pl.program_id(1)
    lab = lab_ref[0, 0]
    masks = [lab == r for r in range(1, NBIN)]
    for r in range(1, NBIN):
        out_ref[b, k, 0, r] = jnp.sum(masks[r - 1].astype(jnp.float32))
    for ch in range(C):
        x = pred_ref[0, ch]
        for r in range(1, NBIN):
            out_ref[b, k, 1 + ch, r] = jnp.sum(jnp.where(masks[r - 1], x, 0.0))


def _epilogue(part_ref, tc_ref, out_ref):
    x = part_ref[...]
    total = jnp.float32(0.0)
    for b in range(B):
        col = jnp.sum(x[b * NS:(b + 1) * NS, :], axis=0)
        def tc_sum(j, r):
            t = jnp.float32(0.0)
            for k in range(KSTEP):
                t = t + tc_ref[b, k, j, r]
            return t

        cnt = [jnp.sum(col[r * L:(r + 1) * L]) + tc_sum(0, r)
               for r in range(NBIN)]
        kmax = jnp.float32(0.0)
        for r in range(1, NBIN):
            kmax = jnp.where(cnt[r] > 0.0, jnp.float32(r), kmax)
        accum = jnp.float32(0.0)
        for r in range(1, NBIN):
            s2 = jnp.float32(0.0)
            for ch in range(C):
                c0 = (1 + ch) * NBIN * L + r * L
                s = jnp.sum(col[c0:c0 + L]) + tc_sum(1 + ch, r)
                s2 = s2 + s * s
            nrm = jnp.sqrt(s2)
            fr = jnp.log(jnp.maximum(3.0 - nrm, 0.0) ** 2 + 1.0)
            valid = (jnp.float32(r) <= kmax).astype(jnp.float32)
            accum = accum + valid * cnt[r] * (fr - LOG10)
        pairs = kmax * (kmax - 1.0) * 0.5
        total = total + pairs * jnp.float32(B * N * LOG10) + (kmax - 1.0) * accum
    out_ref[0, 0] = total


def kernel(pred_similarities, kernel_mask_ndi_labels):
    parts = _binned_sums(pred_similarities, kernel_mask_ndi_labels)
    tc_parts = pl.pallas_call(
        _tc_partial,
        grid=(B, KSTEP),
        in_specs=[
            pl.BlockSpec((1, C, GB, W), lambda b, k: (b, 0, H_SC // GB + k, 0)),
            pl.BlockSpec((1, 1, GB, W), lambda b, k: (b, 0, H_SC // GB + k, 0)),
        ],
        out_shape=jax.ShapeDtypeStruct((B, KSTEP, 1 + C, NBIN), jnp.float32),
        out_specs=pl.BlockSpec(memory_space=pltpu.SMEM),
    )(pred_similarities, kernel_mask_ndi_labels)
    res = pl.pallas_call(
        _epilogue,
        in_specs=[
            pl.BlockSpec(memory_space=pltpu.VMEM),
            pl.BlockSpec(memory_space=pltpu.SMEM),
        ],
        out_shape=jax.ShapeDtypeStruct((1, 1), jnp.float32),
        out_specs=pl.BlockSpec(memory_space=pltpu.SMEM),
    )(parts, tc_parts)
    return res[0, 0]


# speedup vs baseline: 1.0157x; 1.0157x over previous
import functools
import math

import jax
import jax.numpy as jnp
from jax import lax
from jax.experimental import pallas as pl
from jax.experimental.pallas import tpu as pltpu
from jax.experimental.pallas import tpu_sc as plsc

B, C, H, W = 2, 4, 512, 512
N = H * W
NC, NS, L = 2, 16, 16
NW = NC * NS
NBIN = 5
ACC = (1 + C) * NBIN * L
LOG10 = math.log(10.0)

H_SC = 128
GB = 128

_mesh = plsc.VectorSubcoreMesh(
    core_axis_name="c", subcore_axis_name="s", num_cores=NC, num_subcores=NS
)


@functools.partial(
    pl.kernel,
    out_type=jax.ShapeDtypeStruct((NW, ACC), jnp.float32),
    mesh=_mesh,
    scratch_types=[
        pltpu.VMEM((2, H_SC // NS, W), jnp.int32),
        pltpu.VMEM((2, C, H_SC // NS, W), jnp.float32),
        pltpu.VMEM((NBIN * L,), jnp.float32),
        pltpu.VMEM((NBIN * L,), jnp.float32),
        pltpu.VMEM((NBIN * L,), jnp.float32),
        pltpu.VMEM((NBIN * L,), jnp.float32),
        pltpu.VMEM((NBIN * L,), jnp.float32),
        pltpu.VMEM((ACC,), jnp.float32),
        pltpu.SemaphoreType.DMA,
        pltpu.SemaphoreType.DMA,
    ],
    compiler_params=pltpu.CompilerParams(needs_layout_passes=False),
)
def _binned_sums(pred_hbm, lab_hbm, out_hbm, labv, chv, cnt, a0, a1, a2, a3,
                 stage, sem0, sem1):
    cid = lax.axis_index("c")
    sid = lax.axis_index("s")
    batch = cid
    rows = H_SC // NS
    NCHK = 1
    CR = rows // NCHK
    r0 = sid * rows
    accs = [cnt, a0, a1, a2, a3]
    sems = [sem0, sem1]

    zeros = jnp.zeros((L,), jnp.float32)
    for a in accs:
        for j in range(NBIN):
            a[pl.ds(j * L, L)] = zeros

    iota = lax.iota(jnp.int32, L)
    ones = jnp.ones((L,), jnp.float32)
    vecs_per_row = W // L

    def issue(g):
        buf = g & 1
        rb = r0 + g * CR
        hl = pltpu.async_copy(lab_hbm.at[batch, 0, pl.ds(rb, CR), :],
                              labv.at[buf], sems[buf])
        hp = pltpu.async_copy(pred_hbm.at[batch, :, pl.ds(rb, CR), :],
                              chv.at[buf], sems[buf])
        return (hl, hp)

    handles = issue(0)
    for g in range(NCHK):
        nxt = issue(g + 1) if g + 1 < NCHK else None
        for h in handles:
            h.wait()
        handles = nxt
        buf = g & 1

        @plsc.parallel_loop(0, CR * W // L, unroll=8)
        def _(i):
            r = i // vecs_per_row
            col = (i % vecs_per_row) * L
            lab = labv[buf, r, pl.ds(col, L)]
            idx = lab * L + iota
            plsc.addupdate_scatter(cnt, [idx], ones)
            for ch in range(C):
                x = chv[buf, ch, r, pl.ds(col, L)]
                plsc.addupdate_scatter(accs[1 + ch], [idx], x)

    wid = cid * NS + sid
    for j, a in enumerate(accs):
        for k in range(NBIN):
            stage[pl.ds((j * NBIN + k) * L, L)] = a[pl.ds(k * L, L)]
    pltpu.sync_copy(stage, out_hbm.at[wid])


def _tc_partial(pred_ref, lab_ref, out_ref):
    b = pl.program_id(0)
    k = pl.program_id(1)

    @pl.when((b == 0) & (k == 0))
    def _():
        for bb in range(B):
            for j in range(1 + C):
                for r in range(NBIN):
                    out_ref[bb, j, r] = jnp.float32(0.0)

    lab = lab_ref[0, 0]
    masks = [lab == r for r in range(1, NBIN)]
    for r in range(1, NBIN):
        out_ref[b, 0, r] += jnp.sum(masks[r - 1].astype(jnp.float32))
    for ch in range(C):
        x = pred_ref[0, ch]
        for r in range(1, NBIN):
            out_ref[b, 1 + ch, r] += jnp.sum(jnp.where(masks[r - 1], x, 0.0))


def _epilogue(part_ref, tc_ref, out_ref):
    x = part_ref[...]
    total = jnp.float32(0.0)
    for b in range(B):
        col = jnp.sum(x[b * NS:(b + 1) * NS, :], axis=0)
        cnt = [jnp.sum(col[r * L:(r + 1) * L]) + tc_ref[b, 0, r]
               for r in range(NBIN)]
        kmax = jnp.float32(0.0)
        for r in range(1, NBIN):
            kmax = jnp.where(cnt[r] > 0.0, jnp.float32(r), kmax)
        accum = jnp.float32(0.0)
        for r in range(1, NBIN):
            s2 = jnp.float32(0.0)
            for ch in range(C):
                c0 = (1 + ch) * NBIN * L + r * L
                s = jnp.sum(col[c0:c0 + L]) + tc_ref[b, 1 + ch, r]
                s2 = s2 + s * s
            nrm = jnp.sqrt(s2)
            fr = jnp.log(jnp.maximum(3.0 - nrm, 0.0) ** 2 + 1.0)
            valid = (jnp.float32(r) <= kmax).astype(jnp.float32)
            accum = accum + valid * cnt[r] * (fr - LOG10)
        pairs = kmax * (kmax - 1.0) * 0.5
        total = total + pairs * jnp.float32(B * N * LOG10) + (kmax - 1.0) * accum
    out_ref[0, 0] = total


def kernel(pred_similarities, kernel_mask_ndi_labels):
    parts = _binned_sums(pred_similarities, kernel_mask_ndi_labels)
    tc_parts = pl.pallas_call(
        _tc_partial,
        grid=(B, (H - H_SC) // GB),
        in_specs=[
            pl.BlockSpec((1, C, GB, W), lambda b, k: (b, 0, H_SC // GB + k, 0)),
            pl.BlockSpec((1, 1, GB, W), lambda b, k: (b, 0, H_SC // GB + k, 0)),
        ],
        out_shape=jax.ShapeDtypeStruct((B, 1 + C, NBIN), jnp.float32),
        out_specs=pl.BlockSpec(memory_space=pltpu.SMEM),
    )(pred_similarities, kernel_mask_ndi_labels)
    res = pl.pallas_call(
        _epilogue,
        in_specs=[
            pl.BlockSpec(memory_space=pltpu.VMEM),
            pl.BlockSpec(memory_space=pltpu.SMEM),
        ],
        out_shape=jax.ShapeDtypeStruct((1, 1), jnp.float32),
        out_specs=pl.BlockSpec(memory_space=pltpu.SMEM),
    )(parts, tc_parts)
    return res[0, 0]
